# Initial kernel scaffold; baseline (speedup 1.0000x reference)
import jax
import jax.numpy as jnp
from jax.experimental import pallas as pl


def kernel(x, table, W, b):
    raise NotImplementedError("write your pallas kernel here")


# speedup vs baseline: 35.3620x; 35.3620x over previous
import functools

import jax
import jax.numpy as jnp
from jax import lax
from jax.experimental import pallas as pl
from jax.experimental.pallas import tpu as pltpu
from jax.experimental.pallas import tpu_sc as plsc

DICT_SIZE = 1000000
EMBED_DIM = 32
BATCH = 16384
SEQ = 63

NC = 2
NS = 16
NW = NC * NS
B_PER_W = BATCH // NW
PAIRS_PER_W = B_PER_W // 2
NBUF = 2
ITERS = PAIRS_PER_W // NBUF


def _sc_body(x2_hbm, table_hbm, w_hbm, b_hbm, out_hbm,
             idx_v, buf, w_v, b_v, out_v, tp_v, sems):
    wid = lax.axis_index("s") * NC + lax.axis_index("c")
    pair_base = wid * PAIRS_PER_W

    pltpu.sync_copy(x2_hbm.at[pl.ds(pair_base, PAIRS_PER_W)], idx_v)
    pltpu.sync_copy(w_hbm, w_v)
    pltpu.sync_copy(b_hbm, b_v)

    def gather_start(pr, slot):
        pltpu.make_async_copy(
            table_hbm.at[idx_v.at[pr]], buf.at[slot], sems.at[slot]
        ).start()

    def gather_wait(slot):
        pltpu.make_async_copy(
            table_hbm.at[idx_v.at[0]], buf.at[slot], sems.at[slot]
        ).wait()

    for p in range(NBUF):
        gather_start(p, p)

    lanes = lax.iota(jnp.int32, 16)
    bias = b_v[...]

    def step(i, carry):
        phase = lax.rem(i, 4)
        for q in range(NBUF):
            gather_wait(q)
            for bb in range(2):
                acc0 = jnp.zeros((16,), jnp.float32)
                acc1 = jnp.zeros((16,), jnp.float32)
                for s in range(SEQ):
                    r = bb * 64 + s
                    acc0 = acc0 + buf[q, r, 0:16] * w_v[pl.ds(s * 32, 16)]
                    acc1 = acc1 + buf[q, r, 16:32] * w_v[pl.ds(s * 32 + 16, 16)]
                col = phase * 4 + q * 2 + bb
                plsc.store_scatter(tp_v, [lanes, lanes * 0 + col], acc0 + acc1)
            @pl.when(i < ITERS - 1)
            def _():
                gather_start(NBUF * (i + 1) + q, q)

        @pl.when(phase == 3)
        def _():
            z16 = bias
            for r in range(16):
                z16 = z16 + tp_v[r, 0:16]
            out_v[pl.ds((i // 4) * 16, 16)] = 1.0 / (1.0 + jnp.exp(-z16))

        return carry

    lax.fori_loop(0, ITERS, step, jnp.int32(0))

    pltpu.sync_copy(out_v, out_hbm.at[pl.ds(wid * B_PER_W, B_PER_W)])


@functools.partial(jax.jit, static_argnums=())
def kernel(x, table, W, b):
    x64 = jnp.pad(x.astype(jnp.int32), ((0, 0), (0, 1)))
    x2 = x64.reshape(BATCH // 2, 128)
    wv = jnp.reshape(W, (SEQ * EMBED_DIM,)).astype(jnp.float32)
    bv = jnp.broadcast_to(b.astype(jnp.float32), (16,))

    mesh = plsc.VectorSubcoreMesh(
        core_axis_name="c", subcore_axis_name="s",
        num_cores=NC, num_subcores=NS,
    )
    out = pl.kernel(
        _sc_body,
        out_type=jax.ShapeDtypeStruct((BATCH,), jnp.float32),
        mesh=mesh,
        compiler_params=pltpu.CompilerParams(
            needs_layout_passes=False, use_tc_tiling_on_sc=False),
        scratch_types=[
            pltpu.VMEM((PAIRS_PER_W, 128), jnp.int32),
            pltpu.VMEM((NBUF, 128, EMBED_DIM), jnp.float32),
            pltpu.VMEM((SEQ * EMBED_DIM,), jnp.float32),
            pltpu.VMEM((16,), jnp.float32),
            pltpu.VMEM((B_PER_W,), jnp.float32),
            pltpu.VMEM((16, 16), jnp.float32),
            pltpu.SemaphoreType.DMA((NBUF,)),
        ],
    )(x2, table, wv, bv)
    return out.reshape(BATCH, 1)
